# Initial kernel scaffold; baseline (speedup 1.0000x reference)
import jax
import jax.numpy as jnp
from jax.experimental import pallas as pl


def kernel(x, pos_x, pos_y, k):
    raise NotImplementedError("write your pallas kernel here")


# speedup vs baseline: 2.1393x; 2.1393x over previous
import jax
import jax.numpy as jnp
from jax.experimental import pallas as pl
from jax.experimental.pallas import tpu as pltpu

NY = 2048
NX = 16384
BY = 64
KNN = 3


def _knn_kernel(posy_ref, posx_ref, posxT_ref, xT_ref, out_ref):
    yb = posy_ref[...]
    xT = posxT_ref[...]
    x0 = xT[0:1, :]
    x1 = xT[1:2, :]
    x2 = xT[2:3, :]
    y0 = yb[:, 0:1]
    y1 = yb[:, 1:2]
    y2 = yb[:, 2:3]

    ynorm = (y0 * y0 + y2 * y2) + y1 * y1
    xnorm = (x0 * x0 + x2 * x2) + x1 * x1
    dot = jax.lax.dot_general(
        yb, posx_ref[...], (((1,), (1,)), ((), ())),
        preferred_element_type=jnp.float32)
    d2 = (ynorm + xnorm) - 2.0 * dot

    iota = jax.lax.broadcasted_iota(jnp.int32, (1, NX), 1)
    fx0 = xT_ref[0:1, :]
    fx1 = xT_ref[1:2, :]
    fx2 = xT_ref[2:3, :]

    num0 = num1 = num2 = den = None
    BIGF = jnp.float32(1e30)
    for j in range(KNN):
        m = jnp.min(d2, axis=1, keepdims=True)
        is_min = d2 == m
        idx = jnp.min(jnp.where(is_min, iota, NX), axis=1, keepdims=True)
        sel = iota == idx
        selw = sel
        px0 = jnp.sum(jnp.where(selw, x0, 0.0), axis=1, keepdims=True)
        px1 = jnp.sum(jnp.where(selw, x1, 0.0), axis=1, keepdims=True)
        px2 = jnp.sum(jnp.where(selw, x2, 0.0), axis=1, keepdims=True)
        g0 = jnp.sum(jnp.where(selw, fx0, 0.0), axis=1, keepdims=True)
        g1 = jnp.sum(jnp.where(selw, fx1, 0.0), axis=1, keepdims=True)
        g2 = jnp.sum(jnp.where(selw, fx2, 0.0), axis=1, keepdims=True)
        d2 = jnp.where(selw, BIGF, d2)

        dx0 = px0 - y0
        dx1 = px1 - y1
        dx2 = px2 - y2
        d2e = (dx0 * dx0 + dx2 * dx2) + dx1 * dx1
        w = 1.0 / jnp.maximum(d2e, 1e-16)
        if j == 0:
            num0, num1, num2, den = w * g0, w * g1, w * g2, w
        else:
            num0 = num0 + w * g0
            num1 = num1 + w * g1
            num2 = num2 + w * g2
            den = den + w

    out_ref[...] = jnp.concatenate(
        [num0 / den, num1 / den, num2 / den], axis=1)


def kernel(x, pos_x, pos_y, k):
    del k
    posxT = pos_x.T
    xT = x.T
    out = pl.pallas_call(
        _knn_kernel,
        grid=(NY // BY,),
        in_specs=[
            pl.BlockSpec((BY, 3), lambda i: (i, 0)),
            pl.BlockSpec((NX, 3), lambda i: (0, 0)),
            pl.BlockSpec((3, NX), lambda i: (0, 0)),
            pl.BlockSpec((3, NX), lambda i: (0, 0)),
        ],
        out_specs=pl.BlockSpec((BY, 3), lambda i: (i, 0)),
        out_shape=jax.ShapeDtypeStruct((NY, 3), jnp.float32),
    )(pos_y, pos_x, posxT, xT)
    return out


# speedup vs baseline: 6.1552x; 2.8773x over previous
import jax
import jax.numpy as jnp
from jax.experimental import pallas as pl
from jax.experimental.pallas import tpu as pltpu

NY = 2048
NX = 16384
BY = 64
SL = 256
NS = NX // SL
KNN = 3
BIGF = 1e30
BIGI = 2**30


def _knn_kernel(posy_ref, posx_ref, posxT_ref, tblr_ref, out_ref):
    yb = posy_ref[...]
    pxT = posxT_ref[...]
    x0 = pxT[0:1, :]
    x1 = pxT[1:2, :]
    x2 = pxT[2:3, :]
    y0 = yb[:, 0:1]
    y1 = yb[:, 1:2]
    y2 = yb[:, 2:3]

    ynorm = (y0 * y0 + y2 * y2) + y1 * y1
    xnorm = (x0 * x0 + x2 * x2) + x1 * x1
    dot = jax.lax.dot_general(
        yb, posx_ref[...], (((1,), (1,)), ((), ())),
        preferred_element_type=jnp.float32)

    lane = jax.lax.broadcasted_iota(jnp.int32, (1, SL), 1)
    m1 = jnp.full((BY, SL), BIGF, jnp.float32)
    m2 = m1
    m3 = m1
    i1 = jnp.full((BY, SL), BIGI, jnp.int32)
    i2 = i1
    i3 = i1
    for s in range(NS):
        sl = slice(s * SL, (s + 1) * SL)
        v = (ynorm + xnorm[:, sl]) - 2.0 * dot[:, sl]
        gi = lane + (s * SL)
        c1 = v < m1
        dv = jnp.where(c1, m1, v)
        di = jnp.where(c1, i1, gi)
        m1 = jnp.where(c1, v, m1)
        i1 = jnp.where(c1, gi, i1)
        c2 = dv < m2
        dv2 = jnp.where(c2, m2, dv)
        di2 = jnp.where(c2, i2, di)
        m2 = jnp.where(c2, dv, m2)
        i2 = jnp.where(c2, di, i2)
        c3 = dv2 < m3
        m3 = jnp.where(c3, dv2, m3)
        i3 = jnp.where(c3, di2, i3)

    vals = jnp.concatenate([m1, m2, m3], axis=1)
    idxs = jnp.concatenate([i1, i2, i3], axis=1)

    iota128 = jax.lax.broadcasted_iota(jnp.int32, (1, 128), 1)
    tblr = tblr_ref[...]

    num0 = num1 = num2 = den = None
    for j in range(KNN):
        m = jnp.min(vals, axis=1, keepdims=True)
        eq = vals == m
        idxj = jnp.min(jnp.where(eq, idxs, BIGI), axis=1, keepdims=True)
        selpos = eq & (idxs == idxj)
        vals = jnp.where(selpos, BIGF, vals)

        hi = jax.lax.shift_right_logical(idxj, 7)
        lo = jnp.bitwise_and(idxj, 127)
        oh_hi = (iota128 == hi).astype(jnp.float32)
        t2 = jax.lax.dot_general(
            oh_hi, tblr, (((1,), (0,)), ((), ())),
            precision=jax.lax.Precision.HIGHEST,
            preferred_element_type=jnp.float32)
        oh_lo = iota128 == lo

        def pick(c):
            seg = t2[:, c * 128:(c + 1) * 128]
            return jnp.sum(jnp.where(oh_lo, seg, 0.0), axis=1, keepdims=True)

        px0 = pick(0)
        px1 = pick(1)
        px2 = pick(2)
        g0 = pick(3)
        g1 = pick(4)
        g2 = pick(5)

        dx0 = px0 - y0
        dx1 = px1 - y1
        dx2 = px2 - y2
        d2e = (dx0 * dx0 + dx2 * dx2) + dx1 * dx1
        w = 1.0 / jnp.maximum(d2e, 1e-16)
        if j == 0:
            num0, num1, num2, den = w * g0, w * g1, w * g2, w
        else:
            num0 = num0 + w * g0
            num1 = num1 + w * g1
            num2 = num2 + w * g2
            den = den + w

    out_ref[...] = jnp.concatenate(
        [num0 / den, num1 / den, num2 / den], axis=1)


def kernel(x, pos_x, pos_y, k):
    del k
    posxT = pos_x.T
    tbl = jnp.concatenate(
        [pos_x, x, jnp.zeros((NX, 2), jnp.float32)], axis=1)
    tblr = tbl.reshape(128, 128, 8).transpose(0, 2, 1).reshape(128, 1024)
    out = pl.pallas_call(
        _knn_kernel,
        grid=(NY // BY,),
        in_specs=[
            pl.BlockSpec((BY, 3), lambda i: (i, 0)),
            pl.BlockSpec((NX, 3), lambda i: (0, 0)),
            pl.BlockSpec((3, NX), lambda i: (0, 0)),
            pl.BlockSpec((128, 1024), lambda i: (0, 0)),
        ],
        out_specs=pl.BlockSpec((BY, 3), lambda i: (i, 0)),
        out_shape=jax.ShapeDtypeStruct((NY, 3), jnp.float32),
    )(pos_y, pos_x, posxT, tblr)
    return out
